# baseline (speedup 1.0000x reference)
import functools

import jax
import jax.numpy as jnp
from jax import lax
from jax.experimental import pallas as pl
from jax.experimental.pallas import tpu as pltpu
from jax.experimental.pallas import tpu_sc as plsc

D = 64
V = 1000000
RB = 512


@jax.jit
def _embedding_lookup(idx_flat, table):
    info = plsc.get_sparse_core_info()
    nc = info.num_cores
    nw = nc * info.num_subcores
    n_full = V // RB
    tail = V - n_full * RB
    n_loop = n_full // nw
    B0, B1 = 4096, 200
    bpw = B0 // nw

    mesh = plsc.VectorSubcoreMesh(core_axis_name="c", subcore_axis_name="s")
    params = pltpu.CompilerParams(use_tc_tiling_on_sc=True)

    @functools.partial(
        pl.kernel,
        mesh=mesh,
        out_type=jax.ShapeDtypeStruct((V, 128), jnp.float32),
        scratch_types=[
            pltpu.VMEM((RB, D), jnp.float32),
            pltpu.VMEM((RB, 128), jnp.float32),
        ],
        compiler_params=params,
    )
    def widen(table_hbm, t128_hbm, b64, b128):
        wid = lax.axis_index("s") * nc + lax.axis_index("c")

        def do_block(r, n_rows):
            pltpu.sync_copy(table_hbm.at[pl.ds(r, n_rows)], b64.at[pl.ds(0, n_rows)])

            def rows(u, c2):
                for rr in range(8):
                    i = u * 8 + rr
                    for ch in range(D // 16):
                        b128[i, pl.ds(ch * 16, 16)] = b64[i, pl.ds(ch * 16, 16)]
                return c2

            lax.fori_loop(0, n_rows // 8, rows, 0)
            pltpu.sync_copy(b128.at[pl.ds(0, n_rows)], t128_hbm.at[pl.ds(r, n_rows)])

        def body(t, c):
            do_block((t * nw + wid) * RB, RB)
            return c

        lax.fori_loop(0, n_loop, body, 0)

        @pl.when(wid == 0)
        def _():
            do_block((n_loop * nw) * RB, RB)

        @pl.when(wid == 1)
        def _():
            do_block(n_full * RB, tail)

    @functools.partial(
        pl.kernel,
        mesh=mesh,
        out_type=jax.ShapeDtypeStruct((B0, B1, D), jnp.float32),
        scratch_types=[
            pltpu.VMEM((B1,), jnp.int32),
            pltpu.VMEM((B1,), jnp.int32),
            pltpu.VMEM((B1, 128), jnp.float32),
            pltpu.VMEM((B1, 128), jnp.float32),
            pltpu.VMEM((B1, D), jnp.float32),
            pltpu.SemaphoreType.DMA,
            pltpu.SemaphoreType.DMA,
        ],
        compiler_params=params,
    )
    def gather(t128_hbm, idx_hbm, out_hbm, idx0, idx1, g0, g1, b64, s0, s1):
        wid = lax.axis_index("s") * nc + lax.axis_index("c")
        base = wid * bpw
        idx_b = (idx0, idx1)
        g_b = (g0, g1)
        s_b = (s0, s1)

        def start(j, p):
            pltpu.sync_copy(idx_hbm.at[pl.ds((base + j) * B1, B1)], idx_b[p])
            pltpu.make_async_copy(t128_hbm.at[idx_b[p]], g_b[p], s_b[p]).start()

        def finish(j, p):
            pltpu.make_async_copy(t128_hbm.at[idx_b[p]], g_b[p], s_b[p]).wait()
            src = g_b[p]

            def rows(u, c2):
                for rr in range(8):
                    i = u * 8 + rr
                    for ch in range(D // 16):
                        b64[i, pl.ds(ch * 16, 16)] = src[i, pl.ds(ch * 16, 16)]
                return c2

            lax.fori_loop(0, B1 // 8, rows, 0)
            pltpu.sync_copy(b64, out_hbm.at[base + j])

        start(0, 0)

        def body(h, c):
            j0 = 2 * h
            start(j0 + 1, 1)
            finish(j0, 0)

            @pl.when(h < bpw // 2 - 1)
            def _():
                start(j0 + 2, 0)

            finish(j0 + 1, 1)
            return c

        lax.fori_loop(0, bpw // 2, body, 0)

    t128 = widen(table)
    return gather(t128, idx_flat)


def kernel(x, table):
    out = _embedding_lookup(x.reshape(-1), table)
    return out.reshape(x.shape + (D,))
